# baseline (speedup 1.0000x reference)
import functools

import jax
import jax.numpy as jnp
from jax import lax
from jax.experimental import pallas as pl
from jax.experimental.pallas import tpu as pltpu
from jax.experimental.pallas import tpu_sc as plsc

B = 64
C = 192
P = 16
H = 14
W = 14
N = H * W
K = 5
PATCH = 3 * P * P


IPS = 2

SC_ROUNDS = 2 * H


def _patchify_body(x_hbm, out_hbm, bufA, bufB, semA, semB):
    wid = lax.axis_index("s") * 2 + lax.axis_index("c")

    def fire(r, buf, sem):
        b = wid * 2 + r // H
        i = r % H
        for j in range(W):
            pltpu.async_copy(
                x_hbm.at[b, :, pl.ds(16 * i, P), pl.ds(16 * j, P)],
                buf.at[j], sem)

    def drain(buf, sem):
        pltpu.make_async_copy(
            out_hbm.at[0, pl.ds(0, W)], buf, sem).wait()

    def store(r, buf):
        b = wid * 2 + r // H
        i = r % H
        pltpu.sync_copy(buf, out_hbm.at[b, pl.ds(W * i, W)])

    fire(0, bufA, semA)

    def body(t, carry):
        r0 = 2 * t
        fire(r0 + 1, bufB, semB)
        drain(bufA, semA)
        store(r0, bufA)

        @pl.when(t < H - 1)
        def _():
            fire(r0 + 2, bufA, semA)

        drain(bufB, semB)
        store(r0 + 1, bufB)
        return carry

    lax.fori_loop(0, H, body, 0)


_patchify = functools.partial(
    pl.kernel,
    out_type=jax.ShapeDtypeStruct((B, N, 3, P, P), jnp.float32),
    mesh=plsc.VectorSubcoreMesh(core_axis_name="c", subcore_axis_name="s"),
    scratch_types=[
        pltpu.VMEM((W, 3, P, P), jnp.float32),
        pltpu.VMEM((W, 3, P, P), jnp.float32),
        pltpu.SemaphoreType.DMA,
        pltpu.SemaphoreType.DMA,
    ],
    compiler_params=pltpu.CompilerParams(use_tc_tiling_on_sc=False),
)(_patchify_body)


def _gsat_body(patches_ref, pos_ref, Ws_ref, bstem_ref, W1a_ref, W1b_ref,
               be1_ref, w2c_ref, be2_ref, Wm_ref, bm_ref, Wf1_ref, bf1_ref,
               Wf2_ref, bf2_ref, att_ref, gv_ref):
    f32 = jnp.float32
    ri = lax.broadcasted_iota(jnp.int32, (N, N), 0)
    ci = lax.broadcasted_iota(jnp.int32, (N, N), 1)
    for g in range(IPS):
        patches = patches_ref[g]
        nodes = jnp.dot(patches, Ws_ref[...], preferred_element_type=f32)
        nodes = nodes + bstem_ref[...] + pos_ref[...]
        A = jnp.dot(nodes, W1a_ref[...], preferred_element_type=f32)
        Bm = jnp.dot(nodes, W1b_ref[...], preferred_element_type=f32) \
            + be1_ref[...]

        G = lax.dot_general(nodes, nodes, (((1,), (1,)), ((), ())),
                            preferred_element_type=f32)
        nn = nodes * nodes
        sqrow = lax.dot_general(jnp.ones((1, C), f32), nn,
                                (((1,), (1,)), ((), ())),
                                preferred_element_type=f32,
                                precision=lax.Precision.HIGHEST)
        scores = sqrow - 2.0 * G
        scores = jnp.where(ri == ci, f32(1e10), scores)

        agg = jnp.zeros((N, C), f32)
        logits = []
        fsrcs = []
        for _ in range(K):
            m = jnp.min(scores, axis=1, keepdims=True)
            amin = jnp.min(jnp.where(scores == m, ci, N), axis=1,
                           keepdims=True)
            sel = ci == amin
            onehot = jnp.where(sel, f32(1.0), f32(0.0))
            scores = jnp.where(sel, f32(1e30), scores)
            Asrc = jnp.dot(onehot, A, preferred_element_type=f32)
            Fsrc = jnp.dot(onehot, nodes, preferred_element_type=f32)
            e = jnp.maximum(Asrc + Bm, 0.0)
            logits.append(jnp.dot(e, w2c_ref[...],
                                  preferred_element_type=f32))
            fsrcs.append(Fsrc)
        att = 1.0 / (1.0 + jnp.exp(-(jnp.concatenate(logits, axis=1)
                                     + be2_ref[...])))
        for k in range(K):
            agg = agg + lax.slice(att, (0, k), (N, k + 1)) * fsrcs[k]
        att_ref[g] = att

        ne = jnp.maximum(
            jnp.dot(nodes + agg, Wm_ref[...], preferred_element_type=f32)
            + bm_ref[...], 0.0)
        hidden = jax.nn.gelu(
            jnp.dot(ne, Wf1_ref[...], preferred_element_type=f32)
            + bf1_ref[...])
        nm = ne + jnp.dot(hidden, Wf2_ref[...], preferred_element_type=f32) \
            + bf2_ref[...]
        gv_ref[g, 0, :] = jnp.sum(nm, axis=0) * f32(1.0 / N)


def _head_body(g_ref, Wp1_ref, bp1_ref, gamma_ref, beta_ref, wp2r_ref,
               bp2_ref, out_ref):
    f32 = jnp.float32
    p = jnp.dot(g_ref[...], Wp1_ref[...], preferred_element_type=f32) \
        + bp1_ref[...]
    p = jax.nn.gelu(p * gamma_ref[...] + beta_ref[...])
    out_ref[...] = jnp.sum(p * wp2r_ref[...], axis=1, keepdims=True) \
        + bp2_ref[...]


def kernel(x, W_stem, b_stem, pos_embed, W_e1, b_e1, W_e2, b_e2, W_msg, b_msg,
           W_f1, b_f1, W_f2, b_f2, W_p1, b_p1, gamma, beta, W_p2, b_p2):
    f32 = jnp.float32
    patches = _patchify(x).reshape(B, N, PATCH)
    Ws = W_stem.reshape(C, PATCH).T
    pos = pos_embed.transpose(0, 2, 3, 1).reshape(N, C)
    W1a = W_e1[:C]
    W1b = W_e1[C:]

    const2d = lambda: pl.BlockSpec(index_map=lambda i: (0, 0))
    att, gv = pl.pallas_call(
        _gsat_body,
        grid=(B // IPS,),
        in_specs=[
            pl.BlockSpec((IPS, N, PATCH), lambda i: (i, 0, 0)),
            const2d(),
            const2d(),
            const2d(),
            const2d(),
            const2d(),
            const2d(),
            const2d(),
            const2d(),
            const2d(),
            const2d(),
            const2d(),
            const2d(),
            const2d(),
            const2d(),
        ],
        out_specs=[
            pl.BlockSpec((IPS, N, K), lambda i: (i, 0, 0)),
            pl.BlockSpec((IPS, 1, C), lambda i: (i, 0, 0)),
        ],
        out_shape=[
            jax.ShapeDtypeStruct((B, N, K), f32),
            jax.ShapeDtypeStruct((B, 1, C), f32),
        ],
        compiler_params=pltpu.CompilerParams(
            dimension_semantics=("parallel",)),
    )(patches, pos, Ws, b_stem.reshape(1, C), W1a, W1b, b_e1.reshape(1, C),
      W_e2, b_e2.reshape(1, 1), W_msg, b_msg.reshape(1, C),
      W_f1, b_f1.reshape(1, 4 * C), W_f2, b_f2.reshape(1, C))

    pred = pl.pallas_call(
        _head_body,
        out_shape=jax.ShapeDtypeStruct((B, 1), f32),
    )(gv.reshape(B, C), W_p1, b_p1.reshape(1, 1024), gamma.reshape(1, 1024),
      beta.reshape(1, 1024), W_p2.reshape(1, 1024), b_p2.reshape(1, 1))

    return (att.reshape(B * N * K, 1), pred)


# speedup vs baseline: 1.4651x; 1.4651x over previous
import jax
import jax.numpy as jnp
from jax import lax
from jax.experimental import pallas as pl
from jax.experimental.pallas import tpu as pltpu

B = 64
C = 192
P = 16
H = 14
W = 14
N = H * W
K = 5
PATCH = 3 * P * P


IPS = 2

def _gsat_body(patches_ref, pos_ref, Ws_ref, bstem_ref, W1a_ref, W1b_ref,
               be1_ref, w2c_ref, be2_ref, Wm_ref, bm_ref, Wf1_ref, bf1_ref,
               Wf2_ref, bf2_ref, att_ref, gv_ref):
    f32 = jnp.float32
    ri = lax.broadcasted_iota(jnp.int32, (N, N), 0)
    ci = lax.broadcasted_iota(jnp.int32, (N, N), 1)
    for g in range(IPS):
        patches = patches_ref[g]
        nodes = jnp.dot(patches, Ws_ref[...], preferred_element_type=f32)
        nodes = nodes + bstem_ref[...] + pos_ref[...]
        A = jnp.dot(nodes, W1a_ref[...], preferred_element_type=f32)
        Bm = jnp.dot(nodes, W1b_ref[...], preferred_element_type=f32) \
            + be1_ref[...]

        G = lax.dot_general(nodes, nodes, (((1,), (1,)), ((), ())),
                            preferred_element_type=f32)
        nn = nodes * nodes
        sqrow = lax.dot_general(jnp.ones((1, C), f32), nn,
                                (((1,), (1,)), ((), ())),
                                preferred_element_type=f32,
                                precision=lax.Precision.HIGHEST)
        scores = sqrow - 2.0 * G
        scores = jnp.where(ri == ci, f32(1e10), scores)

        agg = jnp.zeros((N, C), f32)
        logits = []
        fsrcs = []
        for _ in range(K):
            m = jnp.min(scores, axis=1, keepdims=True)
            amin = jnp.min(jnp.where(scores == m, ci, N), axis=1,
                           keepdims=True)
            sel = ci == amin
            onehot = jnp.where(sel, f32(1.0), f32(0.0))
            scores = jnp.where(sel, f32(1e30), scores)
            Asrc = jnp.dot(onehot, A, preferred_element_type=f32)
            Fsrc = jnp.dot(onehot, nodes, preferred_element_type=f32)
            e = jnp.maximum(Asrc + Bm, 0.0)
            logits.append(jnp.dot(e, w2c_ref[...],
                                  preferred_element_type=f32))
            fsrcs.append(Fsrc)
        att = 1.0 / (1.0 + jnp.exp(-(jnp.concatenate(logits, axis=1)
                                     + be2_ref[...])))
        for k in range(K):
            agg = agg + lax.slice(att, (0, k), (N, k + 1)) * fsrcs[k]
        att_ref[g] = att

        ne = jnp.maximum(
            jnp.dot(nodes + agg, Wm_ref[...], preferred_element_type=f32)
            + bm_ref[...], 0.0)
        hidden = jax.nn.gelu(
            jnp.dot(ne, Wf1_ref[...], preferred_element_type=f32)
            + bf1_ref[...])
        nm = ne + jnp.dot(hidden, Wf2_ref[...], preferred_element_type=f32) \
            + bf2_ref[...]
        gv_ref[g, 0, :] = jnp.sum(nm, axis=0) * f32(1.0 / N)


def _head_body(g_ref, Wp1_ref, bp1_ref, gamma_ref, beta_ref, wp2r_ref,
               bp2_ref, out_ref):
    f32 = jnp.float32
    p = jnp.dot(g_ref[...], Wp1_ref[...], preferred_element_type=f32) \
        + bp1_ref[...]
    p = jax.nn.gelu(p * gamma_ref[...] + beta_ref[...])
    out_ref[...] = jnp.sum(p * wp2r_ref[...], axis=1, keepdims=True) \
        + bp2_ref[...]


def kernel(x, W_stem, b_stem, pos_embed, W_e1, b_e1, W_e2, b_e2, W_msg, b_msg,
           W_f1, b_f1, W_f2, b_f2, W_p1, b_p1, gamma, beta, W_p2, b_p2):
    f32 = jnp.float32
    Ws = W_stem.reshape(C, PATCH).T
    pos = pos_embed.transpose(0, 2, 3, 1).reshape(N, C)
    W1a = W_e1[:C]
    W1b = W_e1[C:]

    const2d = lambda: pl.BlockSpec(index_map=lambda i: (0, 0))
    HB = B // 2
    gsat = lambda patches: pl.pallas_call(
        _gsat_body,
        grid=(HB // IPS,),
        in_specs=[
            pl.BlockSpec((IPS, N, PATCH), lambda i: (i, 0, 0)),
            const2d(),
            const2d(),
            const2d(),
            const2d(),
            const2d(),
            const2d(),
            const2d(),
            const2d(),
            const2d(),
            const2d(),
            const2d(),
            const2d(),
            const2d(),
            const2d(),
        ],
        out_specs=[
            pl.BlockSpec((IPS, N, K), lambda i: (i, 0, 0)),
            pl.BlockSpec((IPS, 1, C), lambda i: (i, 0, 0)),
        ],
        out_shape=[
            jax.ShapeDtypeStruct((HB, N, K), f32),
            jax.ShapeDtypeStruct((HB, 1, C), f32),
        ],
        compiler_params=pltpu.CompilerParams(
            dimension_semantics=("parallel",)),
    )(patches, pos, Ws, b_stem.reshape(1, C), W1a, W1b, b_e1.reshape(1, C),
      W_e2, b_e2.reshape(1, 1), W_msg, b_msg.reshape(1, C),
      W_f1, b_f1.reshape(1, 4 * C), W_f2, b_f2.reshape(1, C))

    halves = [
        x[h * HB:(h + 1) * HB]
        .reshape(HB, 3, H, P, W, P).transpose(0, 2, 4, 1, 3, 5)
        .reshape(HB, N, PATCH)
        for h in range(2)
    ]
    att0, gv0 = gsat(halves[0])
    att1, gv1 = gsat(halves[1])
    att = jnp.concatenate([att0, att1], axis=0)
    gv = jnp.concatenate([gv0, gv1], axis=0)

    pred = pl.pallas_call(
        _head_body,
        out_shape=jax.ShapeDtypeStruct((B, 1), f32),
    )(gv.reshape(B, C), W_p1, b_p1.reshape(1, 1024), gamma.reshape(1, 1024),
      beta.reshape(1, 1024), W_p2.reshape(1, 1024), b_p2.reshape(1, 1))

    return (att.reshape(B * N * K, 1), pred)


# speedup vs baseline: 2.0456x; 1.3963x over previous
import jax
import jax.numpy as jnp
from jax import lax
from jax.experimental import pallas as pl
from jax.experimental.pallas import tpu as pltpu

B = 64
C = 192
P = 16
H = 14
W = 14
N = H * W
K = 5
PATCH = 3 * P * P


IPS = 4

def _gsat_body(patches_ref, pos_ref, Ws_ref, bstem_ref, W1a_ref, W1b_ref,
               be1_ref, w2c_ref, be2_ref, Wm_ref, bm_ref, Wf1_ref, bf1_ref,
               Wf2_ref, bf2_ref, att_ref, gv_ref):
    f32 = jnp.float32
    ri = lax.broadcasted_iota(jnp.int32, (N, N), 0)
    ci = lax.broadcasted_iota(jnp.int32, (N, N), 1)
    for g in range(IPS):
        patches = patches_ref[g]
        nodes = jnp.dot(patches, Ws_ref[...], preferred_element_type=f32)
        nodes = nodes + bstem_ref[...] + pos_ref[...]
        A = jnp.dot(nodes, W1a_ref[...], preferred_element_type=f32)
        Bm = jnp.dot(nodes, W1b_ref[...], preferred_element_type=f32) \
            + be1_ref[...]

        G = lax.dot_general(nodes, nodes, (((1,), (1,)), ((), ())),
                            preferred_element_type=f32)
        nn = nodes * nodes
        sqrow = lax.dot_general(jnp.ones((1, C), f32), nn,
                                (((1,), (1,)), ((), ())),
                                preferred_element_type=f32,
                                precision=lax.Precision.HIGHEST)
        scores = sqrow - 2.0 * G
        scores = jnp.where(ri == ci, f32(1e10), scores)

        agg = jnp.zeros((N, C), f32)
        logits = []
        fsrcs = []
        for _ in range(K):
            m = jnp.min(scores, axis=1, keepdims=True)
            amin = jnp.min(jnp.where(scores == m, ci, N), axis=1,
                           keepdims=True)
            sel = ci == amin
            onehot = jnp.where(sel, f32(1.0), f32(0.0))
            scores = jnp.where(sel, f32(1e30), scores)
            Asrc = jnp.dot(onehot, A, preferred_element_type=f32)
            Fsrc = jnp.dot(onehot, nodes, preferred_element_type=f32)
            e = jnp.maximum(Asrc + Bm, 0.0)
            logits.append(jnp.dot(e, w2c_ref[...],
                                  preferred_element_type=f32))
            fsrcs.append(Fsrc)
        att = 1.0 / (1.0 + jnp.exp(-(jnp.concatenate(logits, axis=1)
                                     + be2_ref[...])))
        for k in range(K):
            agg = agg + lax.slice(att, (0, k), (N, k + 1)) * fsrcs[k]
        att_ref[g] = att

        ne = jnp.maximum(
            jnp.dot(nodes + agg, Wm_ref[...], preferred_element_type=f32)
            + bm_ref[...], 0.0)
        hidden = jax.nn.gelu(
            jnp.dot(ne, Wf1_ref[...], preferred_element_type=f32)
            + bf1_ref[...])
        nm = ne + jnp.dot(hidden, Wf2_ref[...], preferred_element_type=f32) \
            + bf2_ref[...]
        gv_ref[g, 0, :] = jnp.sum(nm, axis=0) * f32(1.0 / N)


def _head_body(g_ref, Wp1_ref, bp1_ref, gamma_ref, beta_ref, wp2r_ref,
               bp2_ref, out_ref):
    f32 = jnp.float32
    p = jnp.dot(g_ref[...], Wp1_ref[...], preferred_element_type=f32) \
        + bp1_ref[...]
    p = jax.nn.gelu(p * gamma_ref[...] + beta_ref[...])
    out_ref[...] = jnp.sum(p * wp2r_ref[...], axis=1, keepdims=True) \
        + bp2_ref[...]


def kernel(x, W_stem, b_stem, pos_embed, W_e1, b_e1, W_e2, b_e2, W_msg, b_msg,
           W_f1, b_f1, W_f2, b_f2, W_p1, b_p1, gamma, beta, W_p2, b_p2):
    f32 = jnp.float32
    Ws = W_stem.reshape(C, PATCH).T
    pos = pos_embed.transpose(0, 2, 3, 1).reshape(N, C)
    W1a = W_e1[:C]
    W1b = W_e1[C:]

    const2d = lambda: pl.BlockSpec(index_map=lambda i: (0, 0))
    patches = x.reshape(B, 3, H, P, W, P).transpose(0, 2, 4, 1, 3, 5) \
        .reshape(B, N, PATCH)
    att, gv = pl.pallas_call(
        _gsat_body,
        grid=(B // IPS,),
        in_specs=[
            pl.BlockSpec((IPS, N, PATCH), lambda i: (i, 0, 0)),
            const2d(),
            const2d(),
            const2d(),
            const2d(),
            const2d(),
            const2d(),
            const2d(),
            const2d(),
            const2d(),
            const2d(),
            const2d(),
            const2d(),
            const2d(),
            const2d(),
        ],
        out_specs=[
            pl.BlockSpec((IPS, N, K), lambda i: (i, 0, 0)),
            pl.BlockSpec((IPS, 1, C), lambda i: (i, 0, 0)),
        ],
        out_shape=[
            jax.ShapeDtypeStruct((B, N, K), f32),
            jax.ShapeDtypeStruct((B, 1, C), f32),
        ],
        compiler_params=pltpu.CompilerParams(
            dimension_semantics=("parallel",)),
    )(patches, pos, Ws, b_stem.reshape(1, C), W1a, W1b, b_e1.reshape(1, C),
      W_e2, b_e2.reshape(1, 1), W_msg, b_msg.reshape(1, C),
      W_f1, b_f1.reshape(1, 4 * C), W_f2, b_f2.reshape(1, C))

    pred = pl.pallas_call(
        _head_body,
        out_shape=jax.ShapeDtypeStruct((B, 1), f32),
    )(gv.reshape(B, C), W_p1, b_p1.reshape(1, 1024), gamma.reshape(1, 1024),
      beta.reshape(1, 1024), W_p2.reshape(1, 1024), b_p2.reshape(1, 1))

    return (att.reshape(B * N * K, 1), pred)


# speedup vs baseline: 2.0473x; 1.0008x over previous
import jax
import jax.numpy as jnp
from jax import lax
from jax.experimental import pallas as pl
from jax.experimental.pallas import tpu as pltpu

B = 64
C = 192
P = 16
H = 14
W = 14
N = H * W
K = 5
PATCH = 3 * P * P


IPS = 4

def _gsat_body(patches_ref, pos_ref, Ws_ref, bstem_ref, W1a_ref, W1b_ref,
               be1_ref, w2c_ref, be2_ref, Wm_ref, bm_ref, Wf1_ref, bf1_ref,
               Wf2_ref, bf2_ref, att_ref, gv_ref):
    f32 = jnp.float32
    ri = lax.broadcasted_iota(jnp.int32, (N, N), 0)
    ci = lax.broadcasted_iota(jnp.int32, (N, N), 1)
    for g in range(IPS):
        patches = patches_ref[g]
        nodes = jnp.dot(patches, Ws_ref[...], preferred_element_type=f32)
        nodes = nodes + bstem_ref[...] + pos_ref[...]
        A = jnp.dot(nodes, W1a_ref[...], preferred_element_type=f32)
        Bm = jnp.dot(nodes, W1b_ref[...], preferred_element_type=f32) \
            + be1_ref[...]

        G = lax.dot_general(nodes, nodes, (((1,), (1,)), ((), ())),
                            preferred_element_type=f32)
        nn = nodes * nodes
        sqrow = lax.dot_general(jnp.ones((1, C), f32), nn,
                                (((1,), (1,)), ((), ())),
                                preferred_element_type=f32,
                                precision=lax.Precision.HIGHEST)
        scores = sqrow - 2.0 * G
        scores = jnp.where(ri == ci, f32(1e10), scores)

        agg = jnp.zeros((N, C), f32)
        logits = []
        fsrcs = []
        for _ in range(K):
            m = jnp.min(scores, axis=1, keepdims=True)
            amin = jnp.min(jnp.where(scores == m, ci, N), axis=1,
                           keepdims=True)
            sel = ci == amin
            onehot = jnp.where(sel, f32(1.0), f32(0.0))
            scores = jnp.where(sel, f32(1e30), scores)
            Asrc = jnp.dot(onehot, A, preferred_element_type=f32)
            Fsrc = jnp.dot(onehot, nodes, preferred_element_type=f32)
            e = jnp.maximum(Asrc + Bm, 0.0)
            logits.append(jnp.dot(e, w2c_ref[...],
                                  preferred_element_type=f32))
            fsrcs.append(Fsrc)
        att = 1.0 / (1.0 + jnp.exp(-(jnp.concatenate(logits, axis=1)
                                     + be2_ref[...])))
        for k in range(K):
            agg = agg + lax.slice(att, (0, k), (N, k + 1)) * fsrcs[k]
        att_ref[g] = att

        bf16 = jnp.bfloat16
        ne = jnp.maximum(
            jnp.dot((nodes + agg).astype(bf16), Wm_ref[...].astype(bf16),
                    preferred_element_type=f32) + bm_ref[...], 0.0)
        hidden = jax.nn.gelu(
            jnp.dot(ne.astype(bf16), Wf1_ref[...].astype(bf16),
                    preferred_element_type=f32) + bf1_ref[...])
        nm = ne + jnp.dot(hidden.astype(bf16), Wf2_ref[...].astype(bf16),
                          preferred_element_type=f32) + bf2_ref[...]
        gv_ref[g, 0, :] = jnp.sum(nm, axis=0) * f32(1.0 / N)


def _head_body(g_ref, Wp1_ref, bp1_ref, gamma_ref, beta_ref, wp2r_ref,
               bp2_ref, out_ref):
    f32 = jnp.float32
    p = jnp.dot(g_ref[...], Wp1_ref[...], preferred_element_type=f32) \
        + bp1_ref[...]
    p = jax.nn.gelu(p * gamma_ref[...] + beta_ref[...])
    out_ref[...] = jnp.sum(p * wp2r_ref[...], axis=1, keepdims=True) \
        + bp2_ref[...]


def kernel(x, W_stem, b_stem, pos_embed, W_e1, b_e1, W_e2, b_e2, W_msg, b_msg,
           W_f1, b_f1, W_f2, b_f2, W_p1, b_p1, gamma, beta, W_p2, b_p2):
    f32 = jnp.float32
    Ws = W_stem.reshape(C, PATCH).T
    pos = pos_embed.transpose(0, 2, 3, 1).reshape(N, C)
    W1a = W_e1[:C]
    W1b = W_e1[C:]

    const2d = lambda: pl.BlockSpec(index_map=lambda i: (0, 0))
    patches = x.reshape(B, 3, H, P, W, P).transpose(0, 2, 4, 1, 3, 5) \
        .reshape(B, N, PATCH)
    att, gv = pl.pallas_call(
        _gsat_body,
        grid=(B // IPS,),
        in_specs=[
            pl.BlockSpec((IPS, N, PATCH), lambda i: (i, 0, 0)),
            const2d(),
            const2d(),
            const2d(),
            const2d(),
            const2d(),
            const2d(),
            const2d(),
            const2d(),
            const2d(),
            const2d(),
            const2d(),
            const2d(),
            const2d(),
            const2d(),
        ],
        out_specs=[
            pl.BlockSpec((IPS, N, K), lambda i: (i, 0, 0)),
            pl.BlockSpec((IPS, 1, C), lambda i: (i, 0, 0)),
        ],
        out_shape=[
            jax.ShapeDtypeStruct((B, N, K), f32),
            jax.ShapeDtypeStruct((B, 1, C), f32),
        ],
        compiler_params=pltpu.CompilerParams(
            dimension_semantics=("parallel",)),
    )(patches, pos, Ws, b_stem.reshape(1, C), W1a, W1b, b_e1.reshape(1, C),
      W_e2, b_e2.reshape(1, 1), W_msg, b_msg.reshape(1, C),
      W_f1, b_f1.reshape(1, 4 * C), W_f2, b_f2.reshape(1, C))

    pred = pl.pallas_call(
        _head_body,
        out_shape=jax.ShapeDtypeStruct((B, 1), f32),
    )(gv.reshape(B, C), W_p1, b_p1.reshape(1, 1024), gamma.reshape(1, 1024),
      beta.reshape(1, 1024), W_p2.reshape(1, 1024), b_p2.reshape(1, 1))

    return (att.reshape(B * N * K, 1), pred)


# speedup vs baseline: 2.0504x; 1.0015x over previous
import jax
import jax.numpy as jnp
from jax import lax
from jax.experimental import pallas as pl
from jax.experimental.pallas import tpu as pltpu

B = 64
C = 192
P = 16
H = 14
W = 14
N = H * W
K = 5
PATCH = 3 * P * P


IPS = 4

def _gsat_body(patches_ref, pos_ref, Ws_ref, bstem_ref, W1a_ref, W1b_ref,
               be1_ref, w2c_ref, be2_ref, Wm_ref, bm_ref, Wf1_ref, bf1_ref,
               Wf2_ref, bf2_ref, att_ref, gv_ref):
    f32 = jnp.float32
    ri = lax.broadcasted_iota(jnp.int32, (N, N), 0)
    ci = lax.broadcasted_iota(jnp.int32, (N, N), 1)
    for g in range(IPS):
        patches = patches_ref[g]
        nodes = jnp.dot(patches, Ws_ref[...], preferred_element_type=f32)
        nodes = nodes + bstem_ref[...] + pos_ref[...]
        A = jnp.dot(nodes, W1a_ref[...], preferred_element_type=f32)
        Bm = jnp.dot(nodes, W1b_ref[...], preferred_element_type=f32) \
            + be1_ref[...]

        G = lax.dot_general(nodes, nodes, (((1,), (1,)), ((), ())),
                            preferred_element_type=f32)
        nn = nodes * nodes
        sqrow = lax.dot_general(jnp.ones((1, C), f32), nn,
                                (((1,), (1,)), ((), ())),
                                preferred_element_type=f32,
                                precision=lax.Precision.HIGHEST)
        scores = sqrow - 2.0 * G
        scores = jnp.where(ri == ci, f32(1e10), scores)

        agg = jnp.zeros((N, C), f32)
        logits = []
        fsrcs = []
        for _ in range(K):
            m = jnp.min(scores, axis=1, keepdims=True)
            amin = jnp.min(jnp.where(scores == m, ci, N), axis=1,
                           keepdims=True)
            sel = ci == amin
            onehot = jnp.where(sel, f32(1.0), f32(0.0))
            scores = jnp.where(sel, f32(1e30), scores)
            Asrc = jnp.dot(onehot, A, preferred_element_type=f32)
            Fsrc = jnp.dot(onehot, nodes, preferred_element_type=f32)
            e = jnp.maximum(Asrc + Bm, 0.0)
            logits.append(jnp.dot(e, w2c_ref[...],
                                  preferred_element_type=f32))
            fsrcs.append(Fsrc)
        att = 1.0 / (1.0 + jnp.exp(-(jnp.concatenate(logits, axis=1)
                                     + be2_ref[...])))
        for k in range(K):
            agg = agg + lax.slice(att, (0, k), (N, k + 1)) * fsrcs[k]
        att_ref[g] = att

        ne = jnp.maximum(
            jnp.dot(nodes + agg, Wm_ref[...], preferred_element_type=f32)
            + bm_ref[...], 0.0)
        hidden = jax.nn.gelu(
            jnp.dot(ne, Wf1_ref[...], preferred_element_type=f32)
            + bf1_ref[...])
        nm = ne + jnp.dot(hidden, Wf2_ref[...], preferred_element_type=f32) \
            + bf2_ref[...]
        gv_ref[g, 0, :] = jnp.sum(nm, axis=0) * f32(1.0 / N)


def _head_body(g_ref, Wp1_ref, bp1_ref, gamma_ref, beta_ref, wp2r_ref,
               bp2_ref, out_ref):
    f32 = jnp.float32
    p = jnp.dot(g_ref[...], Wp1_ref[...], preferred_element_type=f32) \
        + bp1_ref[...]
    p = jax.nn.gelu(p * gamma_ref[...] + beta_ref[...])
    out_ref[...] = jnp.sum(p * wp2r_ref[...], axis=1, keepdims=True) \
        + bp2_ref[...]


def kernel(x, W_stem, b_stem, pos_embed, W_e1, b_e1, W_e2, b_e2, W_msg, b_msg,
           W_f1, b_f1, W_f2, b_f2, W_p1, b_p1, gamma, beta, W_p2, b_p2):
    f32 = jnp.float32
    Ws = W_stem.reshape(C, PATCH).T
    pos = pos_embed.transpose(0, 2, 3, 1).reshape(N, C)
    W1a = W_e1[:C]
    W1b = W_e1[C:]

    const2d = lambda: pl.BlockSpec(index_map=lambda i: (0, 0))
    patches = x.reshape(B, 3, H, P, W, P).transpose(0, 2, 4, 1, 3, 5) \
        .reshape(B, N, PATCH)
    att, gv = pl.pallas_call(
        _gsat_body,
        grid=(B // IPS,),
        in_specs=[
            pl.BlockSpec((IPS, N, PATCH), lambda i: (i, 0, 0)),
            const2d(),
            const2d(),
            const2d(),
            const2d(),
            const2d(),
            const2d(),
            const2d(),
            const2d(),
            const2d(),
            const2d(),
            const2d(),
            const2d(),
            const2d(),
            const2d(),
        ],
        out_specs=[
            pl.BlockSpec((IPS, N, K), lambda i: (i, 0, 0)),
            pl.BlockSpec((IPS, 1, C), lambda i: (i, 0, 0)),
        ],
        out_shape=[
            jax.ShapeDtypeStruct((B, N, K), f32),
            jax.ShapeDtypeStruct((B, 1, C), f32),
        ],
        compiler_params=pltpu.CompilerParams(
            dimension_semantics=("parallel",)),
    )(patches, pos, Ws, b_stem.reshape(1, C), W1a, W1b, b_e1.reshape(1, C),
      W_e2, b_e2.reshape(1, 1), W_msg, b_msg.reshape(1, C),
      W_f1, b_f1.reshape(1, 4 * C), W_f2, b_f2.reshape(1, C))

    pred = pl.pallas_call(
        _head_body,
        out_shape=jax.ShapeDtypeStruct((B, 1), f32),
    )(gv.reshape(B, C), W_p1, b_p1.reshape(1, 1024), gamma.reshape(1, 1024),
      beta.reshape(1, 1024), W_p2.reshape(1, 1024), b_p2.reshape(1, 1))

    return (att.reshape(B * N * K, 1), pred)
